# baseline (speedup 1.0000x reference)
import functools

import jax
import jax.numpy as jnp
from jax import lax
from jax.experimental import pallas as pl
from jax.experimental.pallas import tpu as pltpu
from jax.experimental.pallas import tpu_sc as plsc

_BM = 2048
_CHUNK = 32


def _copy_body(x_ref, o_ref):
    o_ref[...] = x_ref[...]


def _tc_copy(x):
    m, n = x.shape
    return pl.pallas_call(
        _copy_body,
        grid=(m // _BM,),
        in_specs=[pl.BlockSpec((_BM, n), lambda i: (i, 0))],
        out_specs=pl.BlockSpec((_BM, n), lambda i: (i, 0)),
        out_shape=jax.ShapeDtypeStruct((m, n), x.dtype),
    )(x)


def _sc_copy(x):
    m, n = x.shape
    info = plsc.get_sparse_core_info()
    nc, ns = info.num_cores, info.num_subcores
    nw = nc * ns
    rows_per = m // nw
    n_chunks = rows_per // _CHUNK
    mesh = plsc.VectorSubcoreMesh(core_axis_name="c", subcore_axis_name="s")

    @functools.partial(
        pl.kernel,
        mesh=mesh,
        out_type=jax.ShapeDtypeStruct((m, n), x.dtype),
        scratch_types=[
            pltpu.VMEM((_CHUNK, n), x.dtype),
            pltpu.VMEM((_CHUNK, n), x.dtype),
            pltpu.SemaphoreType.DMA,
            pltpu.SemaphoreType.DMA,
            pltpu.SemaphoreType.DMA,
            pltpu.SemaphoreType.DMA,
        ],
    )
    def k(x_hbm, out_hbm, buf0, buf1, r0, r1, w0, w1):
        wid = lax.axis_index("s") * nc + lax.axis_index("c")
        base = wid * rows_per
        bufs = (buf0, buf1)
        rsems = (r0, r1)
        wsems = (w0, w1)

        def rd(i):
            b = i % 2
            pltpu.async_copy(
                x_hbm.at[pl.ds(base + i * _CHUNK, _CHUNK)], bufs[b], rsems[b]
            )

        def wr(i):
            b = i % 2
            pltpu.async_copy(
                bufs[b], out_hbm.at[pl.ds(base + i * _CHUNK, _CHUNK)], wsems[b]
            )

        rd(0)
        for i in range(n_chunks):
            b = i % 2
            if i + 1 < n_chunks:
                if i >= 1:
                    pltpu.make_async_copy(
                        bufs[1 - b],
                        out_hbm.at[pl.ds(base + (i - 1) * _CHUNK, _CHUNK)],
                        wsems[1 - b],
                    ).wait()
                rd(i + 1)
            pltpu.make_async_copy(
                x_hbm.at[pl.ds(base + i * _CHUNK, _CHUNK)], bufs[b], rsems[b]
            ).wait()
            wr(i)
        for i in (n_chunks - 2, n_chunks - 1):
            b = i % 2
            pltpu.make_async_copy(
                bufs[b],
                out_hbm.at[pl.ds(base + i * _CHUNK, _CHUNK)],
                wsems[b],
            ).wait()

    return k(x)


def kernel(x):
    return (_tc_copy(x), _sc_copy(x))


# speedup vs baseline: 2.3194x; 2.3194x over previous
import jax
import jax.numpy as jnp
from jax.experimental import pallas as pl
from jax.experimental.pallas import tpu as pltpu

_BM = 2048


def _copy_body(x_ref, o_ref):
    o_ref[...] = x_ref[...]


def kernel(x):
    m, n = x.shape
    return pl.pallas_call(
        _copy_body,
        grid=(m // _BM,),
        in_specs=[pl.BlockSpec((_BM, n), lambda i: (i, 0))],
        out_specs=pl.BlockSpec((_BM, n), lambda i: (i, 0)),
        out_shape=jax.ShapeDtypeStruct((m, n), x.dtype),
    )(x)
